# baseline (speedup 1.0000x reference)
import jax
import jax.numpy as jnp
from jax import lax
from jax.experimental import pallas as pl
from jax.experimental.pallas import tpu as pltpu
from jax.experimental.pallas import tpu_sc as plsc

B, N_ID, N_SYM, D = 4096, 200, 200, 64
PAD_IDX = 0
L = 16
NC, NS = 2, 16
NW = NC * NS
TOTAL = B * N_SYM
W = TOTAL // NW
CH = 400
NCHUNK = W // CH
VPC = CH // L
SUB = 80
NSUB = CH // SUB
VPS = SUB // L
PADPOS = B * N_ID


def _body(table, idx_hbm, mask_hbm, out,
          idx_v, mask_v, gidx_v, rows_v, boffs_v, sem):
    wid = lax.axis_index("s") * NC + lax.axis_index("c")
    base = wid * W
    iota = lax.iota(jnp.int32, L)

    for v in range(VPC):
        boffs_v[pl.ds(v * L, L)] = jnp.where(
            iota + (v * L) >= N_SYM, jnp.int32(N_ID), jnp.int32(0))

    def chunk(c, carry):
        start = base + c * CH
        pltpu.sync_copy(idx_hbm.at[pl.ds(start, CH)], idx_v)
        pltpu.sync_copy(mask_hbm.at[pl.ds(start, CH)], mask_v)
        for v in range(VPC):
            m = mask_v[pl.ds(v * L, L)]
            g = idx_v[pl.ds(v * L, L)] + boffs_v[pl.ds(v * L, L)] + start
            g = jnp.where(m != 0, g, jnp.int32(PADPOS))
            gidx_v[v // VPS, pl.ds((v % VPS) * L, L)] = g
        cps = [
            pltpu.async_copy(table.at[gidx_v.at[k]],
                             rows_v.at[pl.ds(k * SUB, SUB)], sem)
            for k in range(NSUB)
        ]
        for cp in cps:
            cp.wait()
        pltpu.sync_copy(rows_v, out.at[pl.ds(start, CH)])
        return carry

    lax.fori_loop(0, NCHUNK, chunk, 0)


@jax.jit
def _run(table_aug, idx_flat, mask_flat):
    f = pl.kernel(
        _body,
        mesh=plsc.VectorSubcoreMesh(core_axis_name="c", subcore_axis_name="s"),
        out_type=jax.ShapeDtypeStruct((TOTAL, D), jnp.float32),
        scratch_types=[
            pltpu.VMEM((CH,), jnp.int32),
            pltpu.VMEM((CH,), jnp.int32),
            pltpu.VMEM((NSUB, SUB), jnp.int32),
            pltpu.VMEM((CH, D), jnp.float32),
            pltpu.VMEM((CH,), jnp.int32),
            pltpu.SemaphoreType.DMA,
        ],
        compiler_params=pltpu.CompilerParams(use_tc_tiling_on_sc=False),
    )
    return f(table_aug, idx_flat, mask_flat)


def kernel(encoded_identifiers, identifiers_idxs_of_all_symbols,
           identifiers_idxs_of_all_symbols_mask, special_words_embedding):
    table = encoded_identifiers.reshape(B * N_ID, D)
    pad_rows = jnp.broadcast_to(special_words_embedding[PAD_IDX], (8, D))
    table_aug = jnp.concatenate([table, pad_rows], axis=0)
    idx_flat = identifiers_idxs_of_all_symbols.reshape(-1).astype(jnp.int32)
    mask_flat = (identifiers_idxs_of_all_symbols_mask
                 .reshape(-1).astype(jnp.int32))
    out = _run(table_aug, idx_flat, mask_flat)
    return out.reshape(B, N_SYM, D)


# speedup vs baseline: 1.0029x; 1.0029x over previous
import jax
import jax.numpy as jnp
from jax import lax
from jax.experimental import pallas as pl
from jax.experimental.pallas import tpu as pltpu
from jax.experimental.pallas import tpu_sc as plsc

B, N_ID, N_SYM, D = 4096, 200, 200, 64
PAD_IDX = 0
L = 16
NC, NS = 2, 16
NW = NC * NS
TOTAL = B * N_SYM
W = TOTAL // NW
CH = 256
NCHUNK = W // CH
VPC = CH // L
SUB = 128
NSUB = CH // SUB
NB = 4
PADPOS = B * N_ID


def _body(table, idx_hbm, mask_hbm, out,
          idx_v, mask_v, gbufs, rbufs, sem_g, sem_w):
    wid = lax.axis_index("s") * NC + lax.axis_index("c")
    base = wid * W
    iota = lax.iota(jnp.int32, L)

    pltpu.sync_copy(idx_hbm.at[pl.ds(base, W)], idx_v)
    pltpu.sync_copy(mask_hbm.at[pl.ds(base, W)], mask_v)

    def compute_gidx(c, gb):
        off = c * CH
        for v in range(VPC):
            o = off + v * L
            p = base + o + iota
            m = mask_v[pl.ds(o, L)]
            g = p - lax.rem(p, jnp.int32(N_SYM)) + idx_v[pl.ds(o, L)]
            gb[pl.ds(v * L, L)] = jnp.where(m != 0, g, jnp.int32(PADPOS))

    def issue_gather(c, gb, rb):
        for k in range(NSUB):
            pltpu.async_copy(table.at[gb.at[pl.ds(k * SUB, SUB)]],
                             rb.at[pl.ds(k * SUB, SUB)], sem_g)

    def wait_gather(rb):
        pltpu.make_async_copy(out.at[pl.ds(0, CH)], rb, sem_g).wait()

    def issue_write(c, rb):
        pltpu.async_copy(rb, out.at[pl.ds(base + c * CH, CH)], sem_w)

    def wait_write(rb):
        pltpu.make_async_copy(rb, out.at[pl.ds(0, CH)], sem_w).wait()

    for c in range(NB - 1):
        compute_gidx(c, gbufs[c])
        issue_gather(c, gbufs[c], rbufs[c])

    def step(cc, carry):
        for par in range(NB):
            c = cc * NB + par
            nxt = (par + NB - 1) % NB

            @pl.when(c + NB - 1 < NCHUNK)
            def _():
                compute_gidx(c + NB - 1, gbufs[nxt])

            @pl.when(c >= 1)
            def _():
                wait_write(rbufs[nxt])

            @pl.when(c + NB - 1 < NCHUNK)
            def _():
                issue_gather(c + NB - 1, gbufs[nxt], rbufs[nxt])

            wait_gather(rbufs[par])
            issue_write(c, rbufs[par])
        return carry

    lax.fori_loop(0, NCHUNK // NB, step, 0)
    wait_write(rbufs[0])


@jax.jit
def _run(table_aug, idx_flat, mask_flat):
    f = pl.kernel(
        _body,
        mesh=plsc.VectorSubcoreMesh(core_axis_name="c", subcore_axis_name="s"),
        out_type=jax.ShapeDtypeStruct((TOTAL, D), jnp.float32),
        scratch_types=[
            pltpu.VMEM((W,), jnp.int32),
            pltpu.VMEM((W,), jnp.int32),
            [pltpu.VMEM((CH,), jnp.int32) for _ in range(NB)],
            [pltpu.VMEM((CH, D), jnp.float32) for _ in range(NB)],
            pltpu.SemaphoreType.DMA,
            pltpu.SemaphoreType.DMA,
        ],
        compiler_params=pltpu.CompilerParams(use_tc_tiling_on_sc=False),
    )
    return f(table_aug, idx_flat, mask_flat)


def kernel(encoded_identifiers, identifiers_idxs_of_all_symbols,
           identifiers_idxs_of_all_symbols_mask, special_words_embedding):
    table = encoded_identifiers.reshape(B * N_ID, D)
    pad_rows = jnp.broadcast_to(special_words_embedding[PAD_IDX], (8, D))
    table_aug = jnp.concatenate([table, pad_rows], axis=0)
    idx_flat = identifiers_idxs_of_all_symbols.reshape(-1).astype(jnp.int32)
    mask_flat = (identifiers_idxs_of_all_symbols_mask
                 .reshape(-1).astype(jnp.int32))
    out = _run(table_aug, idx_flat, mask_flat)
    return out.reshape(B, N_SYM, D)


# speedup vs baseline: 3.2361x; 3.2268x over previous
import jax
import jax.numpy as jnp
from jax import lax
from jax.experimental import pallas as pl
from jax.experimental.pallas import tpu as pltpu
from jax.experimental.pallas import tpu_sc as plsc

B, N_ID, N_SYM, D = 4096, 200, 200, 64
PAD_IDX = 0
L = 16
NC, NS = 2, 16
NW = NC * NS
BPW = B // NW
RPW = BPW * N_SYM
TW = N_ID * D
OW = N_SYM * D
NGRP = 13


def _body(table, pad, idx_hbm, mask_hbm, out,
          idx_v, mask_v, tbufs, obufs, sem_t, sem_w):
    wid = lax.axis_index("s") * NC + lax.axis_index("c")
    tbase = wid * BPW * TW
    obase = wid * BPW * OW
    ibase = wid * RPW
    iota = lax.iota(jnp.int32, L)

    pltpu.sync_copy(idx_hbm.at[pl.ds(ibase, RPW)], idx_v)
    pltpu.sync_copy(mask_hbm.at[pl.ds(ibase, RPW)], mask_v)
    for par in range(2):
        pltpu.sync_copy(pad, tbufs[par].at[pl.ds(TW, D)])

    def fetch(b, par):
        pltpu.async_copy(table.at[pl.ds(tbase + b * TW, TW)],
                         tbufs[par].at[pl.ds(0, TW)], sem_t[par])

    def wait_fetch(par):
        pltpu.make_async_copy(table.at[pl.ds(0, TW)],
                              tbufs[par].at[pl.ds(0, TW)], sem_t[par]).wait()

    def put(b, par):
        pltpu.async_copy(obufs[par], out.at[pl.ds(obase + b * OW, OW)],
                         sem_w[par])

    def wait_put(par):
        pltpu.make_async_copy(obufs[par], out.at[pl.ds(0, OW)],
                              sem_w[par]).wait()

    def compute(b, par):
        tb, ob = tbufs[par], obufs[par]
        row0 = b * N_SYM

        def grp(g, carry):
            gs = jnp.minimum(g * L, jnp.int32(N_SYM - L))
            av = row0 + gs + iota
            idx16 = plsc.load_gather(idx_v, [av])
            m16 = plsc.load_gather(mask_v, [av])
            r = jnp.where(m16 != 0, idx16, jnp.int32(N_ID)) << 6
            o = (gs + iota) << 6
            for w in range(D):
                val = plsc.load_gather(tb, [r + w])
                plsc.store_scatter(ob, [o + w], val)
            return carry

        lax.fori_loop(0, NGRP, grp, 0)

    for par in range(2):
        fetch(par, par)

    def step(cc, carry):
        for par in range(2):
            b = cc * 2 + par
            wait_fetch(par)
            @pl.when(b >= 2)
            def _():
                wait_put(par)
            compute(b, par)
            put(b, par)
            @pl.when(b + 2 < BPW)
            def _():
                fetch(b + 2, par)
        return carry

    lax.fori_loop(0, BPW // 2, step, 0)
    for par in range(2):
        wait_put(par)


@jax.jit
def _run(table_flat, pad_row, idx_flat, mask_flat):
    f = pl.kernel(
        _body,
        mesh=plsc.VectorSubcoreMesh(core_axis_name="c", subcore_axis_name="s"),
        out_type=jax.ShapeDtypeStruct((B * N_SYM * D,), jnp.float32),
        scratch_types=[
            pltpu.VMEM((RPW,), jnp.int32),
            pltpu.VMEM((RPW,), jnp.int32),
            [pltpu.VMEM((TW + D,), jnp.float32) for _ in range(2)],
            [pltpu.VMEM((OW,), jnp.float32) for _ in range(2)],
            [pltpu.SemaphoreType.DMA for _ in range(2)],
            [pltpu.SemaphoreType.DMA for _ in range(2)],
        ],
        compiler_params=pltpu.CompilerParams(use_tc_tiling_on_sc=False,
                                             needs_layout_passes=False),
    )
    return f(table_flat, pad_row, idx_flat, mask_flat)


def kernel(encoded_identifiers, identifiers_idxs_of_all_symbols,
           identifiers_idxs_of_all_symbols_mask, special_words_embedding):
    table_flat = encoded_identifiers.reshape(-1)
    pad_row = special_words_embedding[PAD_IDX]
    idx_flat = identifiers_idxs_of_all_symbols.reshape(-1).astype(jnp.int32)
    mask_flat = (identifiers_idxs_of_all_symbols_mask
                 .reshape(-1).astype(jnp.int32))
    out = _run(table_flat, pad_row, idx_flat, mask_flat)
    return out.reshape(B, N_SYM, D)


# speedup vs baseline: 8.3686x; 2.5860x over previous
import jax
import jax.numpy as jnp
from jax import lax
from jax.experimental import pallas as pl
from jax.experimental.pallas import tpu as pltpu
from jax.experimental.pallas import tpu_sc as plsc

B, N_ID, N_SYM, D = 4096, 200, 200, 64
PAD_IDX = 0
L = 16
NC, NS = 2, 16
NW = NC * NS
BPW = B // NW
RPW = BPW * N_SYM
TW = N_ID * D
OW = N_SYM * D
NGRP = 13


def _body(table, pad, idx_hbm, mask_hbm, out,
          idx_v, mask_v, tbufs, obufs, sem_t, sem_w):
    wid = lax.axis_index("s") * NC + lax.axis_index("c")
    tbase = wid * BPW * TW
    obase = wid * BPW * OW
    ibase = wid * RPW
    iota = lax.iota(jnp.int32, L)

    pltpu.sync_copy(idx_hbm.at[pl.ds(ibase, RPW)], idx_v.at[pl.ds(0, RPW)])
    pltpu.sync_copy(mask_hbm.at[pl.ds(ibase, RPW)], mask_v.at[pl.ds(0, RPW)])
    for par in range(2):
        pltpu.sync_copy(pad, tbufs[par].at[pl.ds(TW, D)])

    def fetch(b, par):
        pltpu.async_copy(table.at[pl.ds(tbase + b * TW, TW)],
                         tbufs[par].at[pl.ds(0, TW)], sem_t[par])

    def wait_fetch(par):
        pltpu.make_async_copy(table.at[pl.ds(0, TW)],
                              tbufs[par].at[pl.ds(0, TW)], sem_t[par]).wait()

    def put(b, par):
        pltpu.async_copy(obufs[par], out.at[pl.ds(obase + b * OW, OW)],
                         sem_w[par])

    def wait_put(par):
        pltpu.make_async_copy(obufs[par], out.at[pl.ds(0, OW)],
                              sem_w[par]).wait()

    def compute(b, par):
        tb, ob = tbufs[par], obufs[par]
        row0 = b * N_SYM

        @plsc.parallel_loop(0, N_SYM, 1, unroll=4)
        def rowloop(s):
            idx_s = idx_v[pl.ds(row0 + s, L)][0]
            m_s = mask_v[pl.ds(row0 + s, L)][0]
            r = jnp.where(m_s != 0, idx_s, jnp.int32(N_ID)) << 6
            o = s << 6
            for k in range(0, D, L):
                ob[pl.ds(o + k, L)] = tb[pl.ds(r + k, L)]

    for par in range(2):
        fetch(par, par)

    def step(cc, carry):
        for par in range(2):
            b = cc * 2 + par
            wait_fetch(par)
            @pl.when(b >= 2)
            def _():
                wait_put(par)
            compute(b, par)
            put(b, par)
            @pl.when(b + 2 < BPW)
            def _():
                fetch(b + 2, par)
        return carry

    lax.fori_loop(0, BPW // 2, step, 0)
    for par in range(2):
        wait_put(par)


@jax.jit
def _run(table_flat, pad_row, idx_flat, mask_flat):
    f = pl.kernel(
        _body,
        mesh=plsc.VectorSubcoreMesh(core_axis_name="c", subcore_axis_name="s"),
        out_type=jax.ShapeDtypeStruct((B * N_SYM * D,), jnp.float32),
        scratch_types=[
            pltpu.VMEM((RPW + L,), jnp.int32),
            pltpu.VMEM((RPW + L,), jnp.int32),
            [pltpu.VMEM((TW + D,), jnp.float32) for _ in range(2)],
            [pltpu.VMEM((OW,), jnp.float32) for _ in range(2)],
            [pltpu.SemaphoreType.DMA for _ in range(2)],
            [pltpu.SemaphoreType.DMA for _ in range(2)],
        ],
        compiler_params=pltpu.CompilerParams(use_tc_tiling_on_sc=False,
                                             needs_layout_passes=False),
    )
    return f(table_flat, pad_row, idx_flat, mask_flat)


def kernel(encoded_identifiers, identifiers_idxs_of_all_symbols,
           identifiers_idxs_of_all_symbols_mask, special_words_embedding):
    table_flat = encoded_identifiers.reshape(-1)
    pad_row = special_words_embedding[PAD_IDX]
    idx_flat = identifiers_idxs_of_all_symbols.reshape(-1).astype(jnp.int32)
    mask_flat = (identifiers_idxs_of_all_symbols_mask
                 .reshape(-1).astype(jnp.int32))
    out = _run(table_flat, pad_row, idx_flat, mask_flat)
    return out.reshape(B, N_SYM, D)


# speedup vs baseline: 8.4579x; 1.0107x over previous
import jax
import jax.numpy as jnp
from jax import lax
from jax.experimental import pallas as pl
from jax.experimental.pallas import tpu as pltpu
from jax.experimental.pallas import tpu_sc as plsc

B, N_ID, N_SYM, D = 4096, 200, 200, 64
PAD_IDX = 0
L = 16
NC, NS = 2, 16
NW = NC * NS
BPW = B // NW
TW = N_ID * D
OW = N_SYM * D
NB = 4
IW = N_SYM + L


def _body(table, pad, idx_hbm, mask_hbm, out,
          ibufs, mbufs, tbufs, obufs, sem_t, sem_w):
    wid = lax.axis_index("s") * NC + lax.axis_index("c")
    tbase = wid * BPW * TW
    obase = wid * BPW * OW
    ibase = wid * BPW * N_SYM

    for par in range(NB):
        pltpu.sync_copy(pad, tbufs[par].at[pl.ds(TW, D)])

    def fetch(b, par):
        pltpu.async_copy(table.at[pl.ds(tbase + b * TW, TW)],
                         tbufs[par].at[pl.ds(0, TW)], sem_t[par])
        pltpu.async_copy(idx_hbm.at[pl.ds(ibase + b * N_SYM, N_SYM)],
                         ibufs[par].at[pl.ds(0, N_SYM)], sem_t[par])
        pltpu.async_copy(mask_hbm.at[pl.ds(ibase + b * N_SYM, N_SYM)],
                         mbufs[par].at[pl.ds(0, N_SYM)], sem_t[par])

    def wait_fetch(par):
        pltpu.make_async_copy(table.at[pl.ds(0, TW)],
                              tbufs[par].at[pl.ds(0, TW)], sem_t[par]).wait()
        pltpu.make_async_copy(idx_hbm.at[pl.ds(0, N_SYM)],
                              ibufs[par].at[pl.ds(0, N_SYM)], sem_t[par]).wait()
        pltpu.make_async_copy(mask_hbm.at[pl.ds(0, N_SYM)],
                              mbufs[par].at[pl.ds(0, N_SYM)], sem_t[par]).wait()

    def put(b, par):
        pltpu.async_copy(obufs[par], out.at[pl.ds(obase + b * OW, OW)],
                         sem_w[par])

    def wait_put(par):
        pltpu.make_async_copy(obufs[par], out.at[pl.ds(0, OW)],
                              sem_w[par]).wait()

    def compute(par):
        ib, mb, tb, ob = ibufs[par], mbufs[par], tbufs[par], obufs[par]

        @plsc.parallel_loop(0, N_SYM, 1, unroll=4)
        def rowloop(s):
            idx_s = ib[pl.ds(s, L)][0]
            m_s = mb[pl.ds(s, L)][0]
            r = jnp.where(m_s != 0, idx_s, jnp.int32(N_ID)) << 6
            o = s << 6
            for k in range(0, D, L):
                ob[pl.ds(o + k, L)] = tb[pl.ds(r + k, L)]

    for j in range(NB - 1):
        fetch(j, j)

    def step(cc, carry):
        for par in range(NB):
            b = cc * NB + par
            nxt = (par + NB - 1) % NB

            @pl.when(b + NB - 1 < BPW)
            def _():
                fetch(b + NB - 1, nxt)

            wait_fetch(par)

            @pl.when(b >= NB)
            def _():
                wait_put(par)

            compute(par)
            put(b, par)
        return carry

    lax.fori_loop(0, BPW // NB, step, 0)
    for par in range(NB):
        wait_put(par)


@jax.jit
def _run(table_flat, pad_row, idx_flat, mask_flat):
    f = pl.kernel(
        _body,
        mesh=plsc.VectorSubcoreMesh(core_axis_name="c", subcore_axis_name="s"),
        out_type=jax.ShapeDtypeStruct((B * N_SYM * D,), jnp.float32),
        scratch_types=[
            [pltpu.VMEM((IW,), jnp.int32) for _ in range(NB)],
            [pltpu.VMEM((IW,), jnp.int32) for _ in range(NB)],
            [pltpu.VMEM((TW + D,), jnp.float32) for _ in range(NB)],
            [pltpu.VMEM((OW,), jnp.float32) for _ in range(NB)],
            [pltpu.SemaphoreType.DMA for _ in range(NB)],
            [pltpu.SemaphoreType.DMA for _ in range(NB)],
        ],
        compiler_params=pltpu.CompilerParams(use_tc_tiling_on_sc=False,
                                             needs_layout_passes=False),
    )
    return f(table_flat, pad_row, idx_flat, mask_flat)


def kernel(encoded_identifiers, identifiers_idxs_of_all_symbols,
           identifiers_idxs_of_all_symbols_mask, special_words_embedding):
    table_flat = encoded_identifiers.reshape(-1)
    pad_row = special_words_embedding[PAD_IDX]
    idx_flat = identifiers_idxs_of_all_symbols.reshape(-1).astype(jnp.int32)
    mask_flat = (identifiers_idxs_of_all_symbols_mask
                 .reshape(-1).astype(jnp.int32))
    out = _run(table_flat, pad_row, idx_flat, mask_flat)
    return out.reshape(B, N_SYM, D)
